# Initial kernel scaffold; baseline (speedup 1.0000x reference)
import jax
import jax.numpy as jnp
from jax.experimental import pallas as pl


def kernel(x, edge_index, edge_type, basis1, comp1, root1, bias1, basis2, comp2, root2, bias2):
    raise NotImplementedError("write your pallas kernel here")


# speedup vs baseline: 3.4972x; 3.4972x over previous
import jax
import jax.numpy as jnp
from jax import lax
from jax.experimental import pallas as pl
from jax.experimental.pallas import tpu as pltpu
from jax.experimental.pallas import tpu_sc as plsc

NC = 2
NS = 16
NWORK = NC * NS
LANES = 16
GROUP = 1024
GROUPC = 256


def _sc_agg_call(table, gidx_all, dkey, nrows, npasses, with_counts):
  ew = dkey.shape[1]
  ngroups = ew // GROUP
  rows_per_tile = nrows // NS
  nzchunks = rows_per_tile // GROUPC
  ncchunks = rows_per_tile // GROUP
  assert ew % GROUP == 0 and ew % GROUPC == 0
  assert nrows % (NS * GROUP) == 0

  out_type = [jax.ShapeDtypeStruct((NC, nrows, npasses, LANES), jnp.float32)]
  scratch = [
      pltpu.VMEM_SHARED((nrows, LANES), jnp.float32),
      pltpu.VMEM((GROUP,), jnp.int32),
      pltpu.VMEM((GROUP,), jnp.int32),
      pltpu.VMEM((GROUP, LANES), jnp.float32),
      pltpu.VMEM((GROUPC, LANES), jnp.float32),
      pltpu.VMEM((GROUPC, LANES), jnp.float32),
  ]
  if with_counts:
    out_type = [jax.ShapeDtypeStruct((NC, nrows), jnp.float32)] + out_type
    scratch.append(pltpu.VMEM((rows_per_tile,), jnp.float32))

  mesh = plsc.VectorSubcoreMesh(core_axis_name="c", subcore_axis_name="s",
                                num_cores=NC, num_subcores=NS)

  def body(table_hbm, gall_hbm, dkey_hbm, const_hbm, *rest):
    if with_counts:
      (cnt_hbm, agg_hbm, acc, dk_g, gi_g, rows_v, ones_v, zeros_v,
       cntc_v) = rest
    else:
      agg_hbm, acc, dk_g, gi_g, rows_v, ones_v, zeros_v = rest
    c = lax.axis_index("c")
    s = lax.axis_index("s")
    wid = s * NC + c
    row0 = s * rows_per_tile

    pltpu.sync_copy(const_hbm.at[0], ones_v)
    pltpu.sync_copy(const_hbm.at[1], zeros_v)

    def zero_own_rows():
      for z in range(nzchunks):
        pltpu.sync_copy(zeros_v, acc.at[pl.ds(row0 + z * GROUPC, GROUPC)])

    zero_own_rows()
    plsc.subcore_barrier()

    if with_counts:
      for g in range(ew // GROUPC):
        pltpu.sync_copy(dkey_hbm.at[wid, pl.ds(g * GROUPC, GROUPC)],
                        dk_g.at[pl.ds(0, GROUPC)])
        pltpu.sync_copy(ones_v, acc.at[dk_g.at[pl.ds(0, GROUPC)]], add=True)
      plsc.subcore_barrier()
      lane_iota = lax.iota(jnp.int32, 16)
      lane_zero = jnp.zeros((16,), jnp.int32)
      for z in range(ncchunks):
        pltpu.sync_copy(acc.at[pl.ds(row0 + z * GROUP, GROUP)], rows_v)

        def cbody(j, _):
          vals = plsc.load_gather(rows_v, [lane_iota + j * 16, lane_zero])
          cntc_v[pl.ds(z * GROUP + j * 16, 16)] = vals
          return 0

        lax.fori_loop(0, GROUP // 16, cbody, 0)
      pltpu.sync_copy(cntc_v, cnt_hbm.at[c, pl.ds(row0, rows_per_tile)])
      zero_own_rows()
      plsc.subcore_barrier()

    for p in range(npasses):
      for g in range(ngroups):
        idx = pl.ds(g * GROUP, GROUP)
        pltpu.sync_copy(dkey_hbm.at[wid, idx], dk_g)
        pltpu.sync_copy(gall_hbm.at[p, wid, idx], gi_g)
        pltpu.sync_copy(table_hbm.at[gi_g], rows_v)
        pltpu.sync_copy(rows_v, acc.at[dk_g], add=True)
      plsc.subcore_barrier()
      pltpu.sync_copy(acc.at[pl.ds(row0, rows_per_tile)],
                      agg_hbm.at[c, pl.ds(row0, rows_per_tile), p])
      zero_own_rows()
      plsc.subcore_barrier()

  kern = pl.kernel(
      body,
      out_type=tuple(out_type),
      mesh=mesh,
      compiler_params=pltpu.CompilerParams(use_tc_tiling_on_sc=False,
                                           needs_layout_passes=False),
      scratch_types=tuple(scratch),
  )
  const = jnp.stack([jnp.ones((GROUPC, LANES), jnp.float32),
                     jnp.zeros((GROUPC, LANES), jnp.float32)])
  return kern(table, gidx_all, dkey, const)


def _tc_layer1(agg1, cnt, x_pad, basis1, comp1, root1, bias1,
               basis2, comp2, root2, bias2, np_, bn):
  ncores, r_, _, d_in = agg1.shape
  d_hid = basis1.shape[2]
  d_out = basis2.shape[2]
  nb = np_ // bn

  def body(agg_ref, cnt_ref, x_ref, b1_ref, c1_ref, r1_ref, bb1_ref,
           b2_ref, c2_ref, r2_ref, bb2_ref, hw_ref, out0_ref):
    x = x_ref[...]
    hacc = jnp.dot(x, r1_ref[...], preferred_element_type=jnp.float32)
    hacc = hacc + bb1_ref[...]
    for r in range(r_):
      asm = agg_ref[0, r] + agg_ref[1, r]
      cntr = cnt_ref[0, r] + cnt_ref[1, r]
      norm = (1.0 / jnp.maximum(cntr, 1.0)).reshape(-1, 1)
      w_r = jnp.zeros((d_in, d_hid), jnp.float32)
      for b in range(b1_ref.shape[0]):
        w_r = w_r + c1_ref[r, b] * b1_ref[b]
      hacc = hacc + jnp.dot(asm * norm, w_r,
                            preferred_element_type=jnp.float32)
    h = jnp.maximum(hacc, 0.0)
    for r in range(r_):
      w2_r = jnp.zeros((d_hid, d_out), jnp.float32)
      for b in range(b2_ref.shape[0]):
        w2_r = w2_r + c2_ref[r, b] * b2_ref[b]
      hw_ref[r] = jnp.dot(h, w2_r, preferred_element_type=jnp.float32)
    out0_ref[...] = jnp.dot(h, r2_ref[...],
                            preferred_element_type=jnp.float32) + bb2_ref[...]

  full = lambda shape: pl.BlockSpec(shape, lambda i: (0,) * len(shape))
  grid_spec = pl.GridSpec(
      grid=(nb,),
      in_specs=[
          pl.BlockSpec((ncores, r_, bn, d_in), lambda i: (0, 0, i, 0)),
          pl.BlockSpec((ncores, r_, bn), lambda i: (0, 0, i)),
          pl.BlockSpec((bn, d_in), lambda i: (i, 0)),
          full(basis1.shape), full(comp1.shape), full(root1.shape),
          full((1, d_hid)),
          full(basis2.shape), full(comp2.shape), full(root2.shape),
          full((1, d_out)),
      ],
      out_specs=[
          pl.BlockSpec((r_, bn, d_out), lambda i: (0, i, 0)),
          pl.BlockSpec((bn, d_out), lambda i: (i, 0)),
      ],
  )
  return pl.pallas_call(
      body,
      grid_spec=grid_spec,
      out_shape=[
          jax.ShapeDtypeStruct((r_, np_, d_out), jnp.float32),
          jax.ShapeDtypeStruct((np_, d_out), jnp.float32),
      ],
  )(agg1, cnt, x_pad, basis1, comp1, root1, bias1.reshape(1, -1),
    basis2, comp2, root2, bias2.reshape(1, -1))


def _tc_layer2(agg2, cnt, out0, np_, bn):
  ncores, r_, _, d_out = agg2.shape
  nb = np_ // bn

  def body(agg_ref, cnt_ref, out0_ref, out_ref):
    acc = out0_ref[...]
    for r in range(r_):
      asm = agg_ref[0, r] + agg_ref[1, r]
      cntr = cnt_ref[0, r] + cnt_ref[1, r]
      norm = (1.0 / jnp.maximum(cntr, 1.0)).reshape(-1, 1)
      acc = acc + asm * norm
    out_ref[...] = acc

  grid_spec = pl.GridSpec(
      grid=(nb,),
      in_specs=[
          pl.BlockSpec((ncores, r_, bn, d_out), lambda i: (0, 0, i, 0)),
          pl.BlockSpec((ncores, r_, bn), lambda i: (0, 0, i)),
          pl.BlockSpec((bn, d_out), lambda i: (i, 0)),
      ],
      out_specs=pl.BlockSpec((bn, d_out), lambda i: (i, 0)),
  )
  return pl.pallas_call(
      body,
      grid_spec=grid_spec,
      out_shape=jax.ShapeDtypeStruct((np_, d_out), jnp.float32),
  )(agg2, cnt, out0)


def kernel(x, edge_index, edge_type, basis1, comp1, root1, bias1,
           basis2, comp2, root2, bias2):
  n, d_in = x.shape
  e = edge_index.shape[1]
  r_ = comp1.shape[0]
  d_out = basis2.shape[2]
  c1 = d_in // LANES
  c2 = d_out // LANES

  bn = 512
  np_ = ((n + bn - 1) // bn) * bn
  nrows = r_ * np_
  assert nrows % (NS * GROUP) == 0

  ew = ((e + NWORK - 1) // NWORK + GROUP - 1) // GROUP * GROUP
  epad = NWORK * ew - e
  src = jnp.pad(edge_index[0], (0, epad))
  dst = jnp.pad(edge_index[1], (0, epad), constant_values=n)
  et = jnp.pad(edge_type, (0, epad))

  dkey = (et * np_ + dst).astype(jnp.int32).reshape(NWORK, ew)
  g1 = (src * c1).astype(jnp.int32).reshape(NWORK, ew)
  g1_all = jnp.stack([g1 + p for p in range(c1)])
  g2 = ((et * np_ + src) * c2).astype(jnp.int32).reshape(NWORK, ew)
  g2_all = jnp.stack([g2 + p for p in range(c2)])

  x_cols = x.reshape(n * c1, LANES)

  cnt, agg1 = _sc_agg_call(x_cols, g1_all, dkey, nrows, c1, with_counts=True)
  agg1 = agg1.reshape(NC, r_, np_, c1 * LANES)
  cnt = cnt.reshape(NC, r_, np_)

  x_pad = jnp.pad(x, ((0, np_ - n), (0, 0)))
  hw, out0 = _tc_layer1(agg1, cnt, x_pad, basis1, comp1, root1, bias1,
                        basis2, comp2, root2, bias2, np_, bn)

  hw_cols = hw.reshape(r_ * np_ * c2, LANES)
  (agg2,) = _sc_agg_call(hw_cols, g2_all, dkey, nrows, c2, with_counts=False)
  agg2 = agg2.reshape(NC, r_, np_, c2 * LANES)

  out = _tc_layer2(agg2, cnt, out0, np_, bn)
  return out[:n]


# speedup vs baseline: 4.1283x; 1.1805x over previous
import jax
import jax.numpy as jnp
from jax import lax
from jax.experimental import pallas as pl
from jax.experimental.pallas import tpu as pltpu
from jax.experimental.pallas import tpu_sc as plsc

NC = 2
NS = 16
NWORK = NC * NS
LANES = 16
GROUP = 640
GROUPC = 256


def _sc_agg_call(table, gidx_all, dkey, nrows, npasses, with_counts):
  ew = dkey.shape[1]
  ngroups = ew // GROUP
  rows_per_tile = nrows // NS
  nzchunks = rows_per_tile // GROUPC
  ncchunks = rows_per_tile // GROUP
  assert ew % GROUP == 0 and ew % GROUPC == 0
  assert nrows % (NS * GROUP) == 0

  out_type = [jax.ShapeDtypeStruct((NC, nrows, npasses, LANES), jnp.float32)]
  scratch = [
      pltpu.VMEM_SHARED((nrows, LANES), jnp.float32),
      pltpu.VMEM((ew,), jnp.int32),
      pltpu.VMEM((2, GROUP), jnp.int32),
      pltpu.VMEM((2, GROUP, LANES), jnp.float32),
      pltpu.VMEM((GROUPC, LANES), jnp.float32),
      pltpu.VMEM((GROUPC, LANES), jnp.float32),
      pltpu.SemaphoreType.DMA,
      pltpu.SemaphoreType.DMA,
  ]
  if with_counts:
    out_type = [jax.ShapeDtypeStruct((NC, nrows), jnp.float32)] + out_type
    scratch.append(pltpu.VMEM((rows_per_tile,), jnp.float32))

  mesh = plsc.VectorSubcoreMesh(core_axis_name="c", subcore_axis_name="s",
                                num_cores=NC, num_subcores=NS)

  def body(table_hbm, gall_hbm, dkey_hbm, const_hbm, *rest):
    if with_counts:
      (cnt_hbm, agg_hbm, acc, dk_v, gi_v, rows_v, ones_v, zeros_v,
       sem0, sem1, cntc_v) = rest
    else:
      agg_hbm, acc, dk_v, gi_v, rows_v, ones_v, zeros_v, sem0, sem1 = rest
    sems = (sem0, sem1)
    c = lax.axis_index("c")
    s = lax.axis_index("s")
    wid = s * NC + c
    row0 = s * rows_per_tile

    pltpu.sync_copy(const_hbm.at[0], ones_v)
    pltpu.sync_copy(const_hbm.at[1], zeros_v)
    pltpu.sync_copy(dkey_hbm.at[wid], dk_v)

    def zero_own_rows():
      for z in range(nzchunks):
        pltpu.sync_copy(zeros_v, acc.at[pl.ds(row0 + z * GROUPC, GROUPC)])

    zero_own_rows()
    plsc.subcore_barrier()

    if with_counts:
      for g in range(ew // GROUPC):
        pltpu.sync_copy(ones_v, acc.at[dk_v.at[pl.ds(g * GROUPC, GROUPC)]],
                        add=True)
      plsc.subcore_barrier()
      lane_iota = lax.iota(jnp.int32, 16)
      lane_zero = jnp.zeros((16,), jnp.int32)
      for z in range(ncchunks):
        pltpu.sync_copy(acc.at[pl.ds(row0 + z * GROUP, GROUP)], rows_v.at[0])

        def cbody(j, _):
          vals = plsc.load_gather(rows_v.at[0],
                                  [lane_iota + j * 16, lane_zero])
          cntc_v[pl.ds(z * GROUP + j * 16, 16)] = vals
          return 0

        lax.fori_loop(0, GROUP // 16, cbody, 0)
      pltpu.sync_copy(cntc_v, cnt_hbm.at[c, pl.ds(row0, rows_per_tile)])
      zero_own_rows()
      plsc.subcore_barrier()

    for p in range(npasses):
      pltpu.sync_copy(gall_hbm.at[p, wid, pl.ds(0, GROUP)], gi_v.at[0])
      desc = pltpu.async_copy(table_hbm.at[gi_v.at[0]], rows_v.at[0], sems[0])
      for g in range(ngroups):
        cur = g % 2
        nxt = (g + 1) % 2
        next_desc = None
        if g + 1 < ngroups:
          pltpu.sync_copy(gall_hbm.at[p, wid, pl.ds((g + 1) * GROUP, GROUP)],
                          gi_v.at[nxt])
          next_desc = pltpu.async_copy(table_hbm.at[gi_v.at[nxt]],
                                       rows_v.at[nxt], sems[nxt])
        desc.wait()
        pltpu.sync_copy(rows_v.at[cur],
                        acc.at[dk_v.at[pl.ds(g * GROUP, GROUP)]], add=True)
        desc = next_desc
      plsc.subcore_barrier()
      pltpu.sync_copy(acc.at[pl.ds(row0, rows_per_tile)],
                      agg_hbm.at[c, pl.ds(row0, rows_per_tile), p])
      zero_own_rows()
      plsc.subcore_barrier()

  kern = pl.kernel(
      body,
      out_type=tuple(out_type),
      mesh=mesh,
      compiler_params=pltpu.CompilerParams(use_tc_tiling_on_sc=False,
                                           needs_layout_passes=False),
      scratch_types=tuple(scratch),
  )
  const = jnp.stack([jnp.ones((GROUPC, LANES), jnp.float32),
                     jnp.zeros((GROUPC, LANES), jnp.float32)])
  return kern(table, gidx_all, dkey, const)


def _tc_layer1(agg1, cnt, x_pad, basis1, comp1, root1, bias1,
               basis2, comp2, root2, bias2, np_, bn):
  ncores, r_, _, d_in = agg1.shape
  d_hid = basis1.shape[2]
  d_out = basis2.shape[2]
  nb = np_ // bn

  def body(agg_ref, cnt_ref, x_ref, b1_ref, c1_ref, r1_ref, bb1_ref,
           b2_ref, c2_ref, r2_ref, bb2_ref, hw_ref, out0_ref):
    x = x_ref[...]
    hacc = jnp.dot(x, r1_ref[...], preferred_element_type=jnp.float32)
    hacc = hacc + bb1_ref[...]
    for r in range(r_):
      asm = agg_ref[0, r] + agg_ref[1, r]
      cntr = cnt_ref[0, r] + cnt_ref[1, r]
      norm = (1.0 / jnp.maximum(cntr, 1.0)).reshape(-1, 1)
      w_r = jnp.zeros((d_in, d_hid), jnp.float32)
      for b in range(b1_ref.shape[0]):
        w_r = w_r + c1_ref[r, b] * b1_ref[b]
      hacc = hacc + jnp.dot(asm * norm, w_r,
                            preferred_element_type=jnp.float32)
    h = jnp.maximum(hacc, 0.0)
    for r in range(r_):
      w2_r = jnp.zeros((d_hid, d_out), jnp.float32)
      for b in range(b2_ref.shape[0]):
        w2_r = w2_r + c2_ref[r, b] * b2_ref[b]
      hw_ref[r] = jnp.dot(h, w2_r, preferred_element_type=jnp.float32)
    out0_ref[...] = jnp.dot(h, r2_ref[...],
                            preferred_element_type=jnp.float32) + bb2_ref[...]

  full = lambda shape: pl.BlockSpec(shape, lambda i: (0,) * len(shape))
  grid_spec = pl.GridSpec(
      grid=(nb,),
      in_specs=[
          pl.BlockSpec((ncores, r_, bn, d_in), lambda i: (0, 0, i, 0)),
          pl.BlockSpec((ncores, r_, bn), lambda i: (0, 0, i)),
          pl.BlockSpec((bn, d_in), lambda i: (i, 0)),
          full(basis1.shape), full(comp1.shape), full(root1.shape),
          full((1, d_hid)),
          full(basis2.shape), full(comp2.shape), full(root2.shape),
          full((1, d_out)),
      ],
      out_specs=[
          pl.BlockSpec((r_, bn, d_out), lambda i: (0, i, 0)),
          pl.BlockSpec((bn, d_out), lambda i: (i, 0)),
      ],
  )
  return pl.pallas_call(
      body,
      grid_spec=grid_spec,
      out_shape=[
          jax.ShapeDtypeStruct((r_, np_, d_out), jnp.float32),
          jax.ShapeDtypeStruct((np_, d_out), jnp.float32),
      ],
  )(agg1, cnt, x_pad, basis1, comp1, root1, bias1.reshape(1, -1),
    basis2, comp2, root2, bias2.reshape(1, -1))


def _tc_layer2(agg2, cnt, out0, np_, bn):
  ncores, r_, _, d_out = agg2.shape
  nb = np_ // bn

  def body(agg_ref, cnt_ref, out0_ref, out_ref):
    acc = out0_ref[...]
    for r in range(r_):
      asm = agg_ref[0, r] + agg_ref[1, r]
      cntr = cnt_ref[0, r] + cnt_ref[1, r]
      norm = (1.0 / jnp.maximum(cntr, 1.0)).reshape(-1, 1)
      acc = acc + asm * norm
    out_ref[...] = acc

  grid_spec = pl.GridSpec(
      grid=(nb,),
      in_specs=[
          pl.BlockSpec((ncores, r_, bn, d_out), lambda i: (0, 0, i, 0)),
          pl.BlockSpec((ncores, r_, bn), lambda i: (0, 0, i)),
          pl.BlockSpec((bn, d_out), lambda i: (i, 0)),
      ],
      out_specs=pl.BlockSpec((bn, d_out), lambda i: (i, 0)),
  )
  return pl.pallas_call(
      body,
      grid_spec=grid_spec,
      out_shape=jax.ShapeDtypeStruct((np_, d_out), jnp.float32),
  )(agg2, cnt, out0)


def kernel(x, edge_index, edge_type, basis1, comp1, root1, bias1,
           basis2, comp2, root2, bias2):
  n, d_in = x.shape
  e = edge_index.shape[1]
  r_ = comp1.shape[0]
  d_out = basis2.shape[2]
  c1 = d_in // LANES
  c2 = d_out // LANES

  bn = 512
  np_ = ((n + bn - 1) // bn) * bn
  nrows = r_ * np_
  assert nrows % (NS * GROUP) == 0

  ew = ((e + NWORK - 1) // NWORK + GROUP - 1) // GROUP * GROUP
  epad = NWORK * ew - e
  src = jnp.pad(edge_index[0], (0, epad))
  dst = jnp.pad(edge_index[1], (0, epad), constant_values=n)
  et = jnp.pad(edge_type, (0, epad))

  dkey = (et * np_ + dst).astype(jnp.int32).reshape(NWORK, ew)
  g1 = (src * c1).astype(jnp.int32).reshape(NWORK, ew)
  g1_all = jnp.stack([g1 + p for p in range(c1)])
  g2 = ((et * np_ + src) * c2).astype(jnp.int32).reshape(NWORK, ew)
  g2_all = jnp.stack([g2 + p for p in range(c2)])

  x_cols = x.reshape(n * c1, LANES)

  cnt, agg1 = _sc_agg_call(x_cols, g1_all, dkey, nrows, c1, with_counts=True)
  agg1 = agg1.reshape(NC, r_, np_, c1 * LANES)
  cnt = cnt.reshape(NC, r_, np_)

  x_pad = jnp.pad(x, ((0, np_ - n), (0, 0)))
  hw, out0 = _tc_layer1(agg1, cnt, x_pad, basis1, comp1, root1, bias1,
                        basis2, comp2, root2, bias2, np_, bn)

  hw_cols = hw.reshape(r_ * np_ * c2, LANES)
  (agg2,) = _sc_agg_call(hw_cols, g2_all, dkey, nrows, c2, with_counts=False)
  agg2 = agg2.reshape(NC, r_, np_, c2 * LANES)

  out = _tc_layer2(agg2, cnt, out0, np_, bn)
  return out[:n]
